# baseline (speedup 1.0000x reference)
import functools
import math

import jax
import jax.numpy as jnp
from jax import lax
from jax.experimental import pallas as pl
from jax.experimental.pallas import tpu as pltpu
from jax.experimental.pallas import tpu_sc as plsc

_C = 128
_SC0, _SC1 = 0.1, 0.2
_NC, _NS = 2, 16
_NW = _NC * _NS
_CHUNK = 128
_EPG = _NW * _CHUNK
_BLK = 512


def _mlp_body(x_ref, w_ref, b_ref, o_ref):
    a = x_ref[...]
    for l in range(w_ref.shape[0]):
        a = jnp.dot(a, w_ref[l], preferred_element_type=jnp.float32)
        a = jnp.maximum(a + b_ref[l][None, :], 0.0)
    o_ref[...] = a


def _mlp(x, W, b):
    n = x.shape[0]
    nl = W.shape[0]
    return pl.pallas_call(
        _mlp_body,
        grid=(pl.cdiv(n, _BLK),),
        in_specs=[
            pl.BlockSpec((_BLK, _C), lambda i: (i, 0)),
            pl.BlockSpec((nl, _C, _C), lambda i: (0, 0, 0)),
            pl.BlockSpec((nl, _C), lambda i: (0, 0)),
        ],
        out_specs=pl.BlockSpec((_BLK, _C), lambda i: (i, 0)),
        out_shape=jax.ShapeDtypeStruct((n, _C), jnp.float32),
    )(x, W, b)


def _combine_body(x_ref, a1_ref, a2_ref, ws_ref, w0_ref, w1_ref, o_ref):
    h = jnp.dot(x_ref[...], ws_ref[...], preferred_element_type=jnp.float32)
    h = h + _SC0 * jnp.dot(a1_ref[...], w0_ref[...],
                           preferred_element_type=jnp.float32)
    h = h + _SC1 * jnp.dot(a2_ref[...], w1_ref[...],
                           preferred_element_type=jnp.float32)
    o_ref[...] = jnp.maximum(h, 0.0)


def _combine(x, a1, a2, Wself, W0, W1):
    n = x.shape[0]
    row = pl.BlockSpec((_BLK, _C), lambda i: (i, 0))
    mat = pl.BlockSpec((_C, _C), lambda i: (0, 0))
    return pl.pallas_call(
        _combine_body,
        grid=(pl.cdiv(n, _BLK),),
        in_specs=[row, row, row, mat, mat, mat],
        out_specs=row,
        out_shape=jax.ShapeDtypeStruct((n, _C), jnp.float32),
    )(x, a1, a2, Wself, W0, W1)


def _score_body(x_ref, p_ref, nrm_ref, o_ref):
    t = jnp.dot(x_ref[...], p_ref[...].T,
                preferred_element_type=jnp.float32) / nrm_ref[0, 0]
    o_ref[...] = jnp.broadcast_to(jnp.maximum(t, 0.0), o_ref.shape)


def _score(x, p):
    n = x.shape[0]
    nrm = jnp.linalg.norm(p).reshape(1, 1)
    out = pl.pallas_call(
        _score_body,
        grid=(pl.cdiv(n, _BLK),),
        in_specs=[
            pl.BlockSpec((_BLK, _C), lambda i: (i, 0)),
            pl.BlockSpec((1, _C), lambda i: (0, 0)),
            pl.BlockSpec((1, 1), lambda i: (0, 0), memory_space=pltpu.SMEM),
        ],
        out_specs=pl.BlockSpec((_BLK, _C), lambda i: (i, 0)),
        out_shape=jax.ShapeDtypeStruct((n, _C), jnp.float32),
    )(x, p.reshape(1, _C), nrm)
    return out[:, 0]


@functools.partial(jax.jit, static_argnames=("n_out16",))
def _segsum_sc(table, src2, dst2, ew2, cnt, n_out16):
    rpt = n_out16 // _NS
    mesh = plsc.VectorSubcoreMesh(core_axis_name="c", subcore_axis_name="s")

    @functools.partial(
        pl.kernel,
        mesh=mesh,
        out_type=jax.ShapeDtypeStruct((_NC, n_out16, _C), jnp.float32),
        compiler_params=pltpu.CompilerParams(needs_layout_passes=False),
        scratch_types=[
            pltpu.VMEM((_CHUNK,), jnp.int32),
            pltpu.VMEM((_CHUNK,), jnp.int32),
            pltpu.VMEM((_CHUNK,), jnp.float32),
            pltpu.VMEM((_CHUNK, _C), jnp.float32),
            pltpu.VMEM((16,), jnp.int32),
            pltpu.VMEM_SHARED((n_out16, _C), jnp.float32),
            pltpu.SemaphoreType.DMA,
        ],
    )
    def k(table_h, src_h, dst_h, ew_h, cnt_h, out_h,
          src_v, dst_v, ew_v, rows_v, cnt_v, acc_s, sem):
        cid = lax.axis_index("c")
        sid = lax.axis_index("s")
        wid = sid * _NC + cid

        def zrow(r, c):
            for j in range(_C // 16):
                rows_v[r, pl.ds(j * 16, 16)] = jnp.zeros((16,), jnp.float32)
            return c
        lax.fori_loop(0, _CHUNK, zrow, 0)

        base = sid * rpt
        off = 0
        while off < rpt:
            sz = min(_CHUNK, rpt - off)
            pltpu.sync_copy(rows_v.at[pl.ds(0, sz)],
                            acc_s.at[pl.ds(base + off, sz)])
            off += sz
        plsc.subcore_barrier()

        pltpu.sync_copy(cnt_h, cnt_v)
        n_edges = cnt_v[pl.ds(0, 16)][0]
        n_grp = (n_edges + _EPG - 1) // _EPG

        def body(g, c):
            row = g * _NW + wid
            pltpu.sync_copy(src_h.at[row], src_v)
            pltpu.sync_copy(dst_h.at[row], dst_v)
            pltpu.sync_copy(ew_h.at[row], ew_v)
            pltpu.async_copy(table_h.at[src_v], rows_v, sem).wait()

            def scale(rb, cc):
                wv = ew_v[pl.ds(rb * 16, 16)]
                for j in range(16):
                    w = wv[j]
                    r = rb * 16 + j
                    for cb in range(_C // 16):
                        sl = pl.ds(cb * 16, 16)
                        rows_v[r, sl] = rows_v[r, sl] * w
                return cc
            lax.fori_loop(0, _CHUNK // 16, scale, 0)
            pltpu.sync_copy(rows_v, acc_s.at[dst_v], add=True)
            return c
        lax.fori_loop(0, n_grp, body, 0)
        plsc.subcore_barrier()

        off = 0
        while off < rpt:
            sz = min(_CHUNK, rpt - off)
            pltpu.sync_copy(acc_s.at[pl.ds(base + off, sz)],
                            out_h.at[cid, pl.ds(base + off, sz)])
            off += sz

    return k(table, src2, dst2, ew2, cnt)


def _segsum(table, src2, dst2, ew2, cnt, n_out):
    n_out16 = -(-n_out // (_NS * 8)) * (_NS * 8)
    parts = _segsum_sc(table, src2, dst2, ew2, cnt, n_out16)
    return (parts[0] + parts[1])[:n_out]


@functools.partial(jax.jit, static_argnames=("n_out16", "n_grp"))
def _segsum_map_sc(table, src2, dst2, ew2, mapp, n_out16, n_grp):
    rpt = n_out16 // _NS
    nmapsz = mapp.shape[0]
    mesh = plsc.VectorSubcoreMesh(core_axis_name="c", subcore_axis_name="s")

    @functools.partial(
        pl.kernel,
        mesh=mesh,
        out_type=jax.ShapeDtypeStruct((_NC, n_out16, _C), jnp.float32),
        compiler_params=pltpu.CompilerParams(needs_layout_passes=False),
        scratch_types=[
            pltpu.VMEM((nmapsz,), jnp.int32),
            pltpu.VMEM((_CHUNK,), jnp.int32),
            pltpu.VMEM((_CHUNK,), jnp.int32),
            pltpu.VMEM((_CHUNK,), jnp.float32),
            pltpu.VMEM((_CHUNK,), jnp.int32),
            pltpu.VMEM((_CHUNK,), jnp.int32),
            pltpu.VMEM((_CHUNK, _C), jnp.float32),
            pltpu.VMEM_SHARED((n_out16, _C), jnp.float32),
            pltpu.SemaphoreType.DMA,
        ],
    )
    def k(table_h, src_h, dst_h, ew_h, map_h, out_h,
          map_v, src_v, dst_v, ew_v, ms_v, md_v, rows_v, acc_s, sem):
        cid = lax.axis_index("c")
        sid = lax.axis_index("s")
        wid = sid * _NC + cid

        pltpu.sync_copy(map_h, map_v)

        def zrow(r, c):
            for j in range(_C // 16):
                rows_v[r, pl.ds(j * 16, 16)] = jnp.zeros((16,), jnp.float32)
            return c
        lax.fori_loop(0, _CHUNK, zrow, 0)

        base = sid * rpt
        off = 0
        while off < rpt:
            sz = min(_CHUNK, rpt - off)
            pltpu.sync_copy(rows_v.at[pl.ds(0, sz)],
                            acc_s.at[pl.ds(base + off, sz)])
            off += sz
        plsc.subcore_barrier()

        def body(g, c):
            row = g * _NW + wid
            pltpu.sync_copy(src_h.at[row], src_v)
            pltpu.sync_copy(dst_h.at[row], dst_v)
            pltpu.sync_copy(ew_h.at[row], ew_v)
            for rb in range(_CHUNK // 16):
                sl = pl.ds(rb * 16, 16)
                s = plsc.load_gather(map_v, [src_v[sl]])
                d = plsc.load_gather(map_v, [dst_v[sl]])
                ok = (s >= 0) & (d >= 0)
                ms_v[sl] = jnp.maximum(s, 0)
                md_v[sl] = jnp.maximum(d, 0)
                ew_v[sl] = jnp.where(ok, ew_v[sl], 0.0)
            pltpu.async_copy(table_h.at[ms_v], rows_v, sem).wait()

            def scale(rb, cc):
                wv = ew_v[pl.ds(rb * 16, 16)]
                for j in range(16):
                    w = wv[j]
                    r = rb * 16 + j
                    for cb in range(_C // 16):
                        sl = pl.ds(cb * 16, 16)
                        rows_v[r, sl] = rows_v[r, sl] * w
                return cc
            lax.fori_loop(0, _CHUNK // 16, scale, 0)
            pltpu.sync_copy(rows_v, acc_s.at[md_v], add=True)
            return c
        lax.fori_loop(0, n_grp, body, 0)
        plsc.subcore_barrier()

        off = 0
        while off < rpt:
            sz = min(_CHUNK, rpt - off)
            pltpu.sync_copy(acc_s.at[pl.ds(base + off, sz)],
                            out_h.at[cid, pl.ds(base + off, sz)])
            off += sz

    return k(table, src2, dst2, ew2, mapp)


def _segsum_map(table, src2, dst2, ew2, mapp, n_out):
    n_out16 = -(-n_out // (_NS * 8)) * (_NS * 8)
    n_grp = src2.shape[0] // _NW
    parts = _segsum_map_sc(table, src2, dst2, ew2, mapp, n_out16, n_grp)
    return (parts[0] + parts[1])[:n_out]


def _pad_rows(a, rows):
    pad = rows * _CHUNK - a.shape[0]
    if pad:
        a = jnp.concatenate([a, jnp.zeros((pad,), a.dtype)])
    return a.reshape(rows, _CHUNK)


def _compact_pad(mask, arrs, rows):
    pos = jnp.cumsum(mask.astype(jnp.int32)) - 1
    tgt = jnp.where(mask, pos, rows * _CHUNK)
    outs = []
    for a in arrs:
        z = jnp.zeros((rows * _CHUNK,), a.dtype)
        outs.append(z.at[tgt].set(a, mode="drop").reshape(rows, _CHUNK))
    cnt = jnp.full((16,), jnp.sum(mask, dtype=jnp.int32), jnp.int32)
    return outs, cnt


def _rows_for(n):
    return -(-n // _EPG) * _NW


def kernel(x, edge_index, edge_weight, node_pos, mlp_W, mlp_b,
           mp_Wself, mp_Wmu, pool_p):
    del node_pos
    N0 = x.shape[0]
    E = edge_weight.shape[0]
    k1 = math.ceil(0.5 * N0)
    k2 = math.ceil(0.5 * k1)

    def mus(i, xx, mapp):
        n = xx.shape[0]
        xx = _mlp(xx, mlp_W[i], mlp_b[i])
        for l in range(mp_Wself.shape[1]):
            a1 = _segsum_map(xx, src0_2, dst0_2, ew0_2, mapp, n)
            a2 = _segsum_map(a1, src0_2, dst0_2, ew0_2, mapp, n)
            xx = _combine(xx, a1, a2, mp_Wself[i, l],
                          mp_Wmu[i, l, 0], mp_Wmu[i, l, 1])
        return xx

    src0, dst0 = edge_index[0], edge_index[1]
    rows_e = _rows_for(E)
    src0_2 = _pad_rows(src0, rows_e)
    dst0_2 = _pad_rows(dst0, rows_e)
    ew0_2 = _pad_rows(edge_weight, rows_e)
    ident = jnp.arange(N0, dtype=jnp.int32)

    x0 = mus(0, x, ident)

    score1 = _score(x0, pool_p[0])
    _, perm1 = lax.top_k(score1, k1)
    map1 = jnp.full((N0,), -1, jnp.int32).at[perm1].set(
        jnp.arange(k1, dtype=jnp.int32))

    rows_p1 = _rows_for(k1)
    g_src = _pad_rows(perm1, rows_p1)
    g_dst = _pad_rows(jnp.arange(k1, dtype=jnp.int32), rows_p1)
    g_ew = _pad_rows(score1[perm1], rows_p1)
    cnt_p1 = jnp.full((16,), k1, jnp.int32)
    x1_in = _segsum(x0, g_src, g_dst, g_ew, cnt_p1, k1)

    x1 = mus(1, x1_in, map1)

    score2 = _score(x1, pool_p[1])
    _, perm2 = lax.top_k(score2, k2)
    nmap2 = jnp.full((k1,), -1, jnp.int32).at[perm2].set(
        jnp.arange(k2, dtype=jnp.int32))
    map12 = jnp.where(map1 >= 0, nmap2[jnp.maximum(map1, 0)], -1)
    map4 = jnp.where(map12 >= 0, map1, -1)
    map5 = jnp.where(map12 >= 0, ident, -1)

    rows_p2 = _rows_for(k2)
    g2_src = _pad_rows(perm2, rows_p2)
    g2_dst = _pad_rows(jnp.arange(k2, dtype=jnp.int32), rows_p2)
    g2_ew = _pad_rows(score2[perm2], rows_p2)
    cnt_p2 = jnp.full((16,), k2, jnp.int32)
    x2_in = _segsum(x1, g2_src, g2_dst, g2_ew, cnt_p2, k2)

    x2 = mus(2, x2_in, map12)
    x3 = mus(3, x2, map12)

    rows_u1 = _rows_for(k2)
    u1_src = _pad_rows(jnp.arange(k2, dtype=jnp.int32), rows_u1)
    u1_dst = _pad_rows(perm2, rows_u1)
    u1_ew = _pad_rows(jnp.ones((k2,), jnp.float32), rows_u1)
    x4_in = _segsum(x3, u1_src, u1_dst, u1_ew, cnt_p2, k1)
    x4 = mus(4, x4_in, map4)

    rows_u0 = _rows_for(k1)
    u0_src = _pad_rows(jnp.arange(k1, dtype=jnp.int32), rows_u0)
    u0_dst = _pad_rows(perm1, rows_u0)
    u0_ew = _pad_rows(jnp.ones((k1,), jnp.float32), rows_u0)
    x5_in = _segsum(x4, u0_src, u0_dst, u0_ew, cnt_p1, N0)
    x5 = mus(5, x5_in, map5)
    return x5


# speedup vs baseline: 7.1284x; 7.1284x over previous
import functools
import math

import jax
import jax.numpy as jnp
from jax import lax
from jax.experimental import pallas as pl
from jax.experimental.pallas import tpu as pltpu
from jax.experimental.pallas import tpu_sc as plsc

_C = 128
_SC0, _SC1 = 0.1, 0.2
_NC, _NS = 2, 16
_NW = _NC * _NS
_CHUNK = 128
_EPG = _NW * _CHUNK
_BLK = 512


def _mlp_body(x_ref, w_ref, b_ref, o_ref):
    a = x_ref[...]
    for l in range(w_ref.shape[0]):
        a = jnp.dot(a, w_ref[l], preferred_element_type=jnp.float32)
        a = jnp.maximum(a + b_ref[l][None, :], 0.0)
    o_ref[...] = a


def _mlp(x, W, b):
    n = x.shape[0]
    nl = W.shape[0]
    return pl.pallas_call(
        _mlp_body,
        grid=(pl.cdiv(n, _BLK),),
        in_specs=[
            pl.BlockSpec((_BLK, _C), lambda i: (i, 0)),
            pl.BlockSpec((nl, _C, _C), lambda i: (0, 0, 0)),
            pl.BlockSpec((nl, _C), lambda i: (0, 0)),
        ],
        out_specs=pl.BlockSpec((_BLK, _C), lambda i: (i, 0)),
        out_shape=jax.ShapeDtypeStruct((n, _C), jnp.float32),
    )(x, W, b)


def _combine_body(x_ref, a1_ref, a2_ref, ws_ref, w0_ref, w1_ref, o_ref):
    h = jnp.dot(x_ref[...], ws_ref[...], preferred_element_type=jnp.float32)
    h = h + _SC0 * jnp.dot(a1_ref[...], w0_ref[...],
                           preferred_element_type=jnp.float32)
    h = h + _SC1 * jnp.dot(a2_ref[...], w1_ref[...],
                           preferred_element_type=jnp.float32)
    o_ref[...] = jnp.maximum(h, 0.0)


def _combine(x, a1, a2, Wself, W0, W1):
    n = x.shape[0]
    row = pl.BlockSpec((_BLK, _C), lambda i: (i, 0))
    mat = pl.BlockSpec((_C, _C), lambda i: (0, 0))
    return pl.pallas_call(
        _combine_body,
        grid=(pl.cdiv(n, _BLK),),
        in_specs=[row, row, row, mat, mat, mat],
        out_specs=row,
        out_shape=jax.ShapeDtypeStruct((n, _C), jnp.float32),
    )(x, a1, a2, Wself, W0, W1)


def _score_body(x_ref, p_ref, nrm_ref, o_ref):
    t = jnp.dot(x_ref[...], p_ref[...].T,
                preferred_element_type=jnp.float32) / nrm_ref[0, 0]
    o_ref[...] = jnp.broadcast_to(jnp.maximum(t, 0.0), o_ref.shape)


def _score(x, p):
    n = x.shape[0]
    nrm = jnp.linalg.norm(p).reshape(1, 1)
    out = pl.pallas_call(
        _score_body,
        grid=(pl.cdiv(n, _BLK),),
        in_specs=[
            pl.BlockSpec((_BLK, _C), lambda i: (i, 0)),
            pl.BlockSpec((1, _C), lambda i: (0, 0)),
            pl.BlockSpec((1, 1), lambda i: (0, 0), memory_space=pltpu.SMEM),
        ],
        out_specs=pl.BlockSpec((_BLK, _C), lambda i: (i, 0)),
        out_shape=jax.ShapeDtypeStruct((n, _C), jnp.float32),
    )(x, p.reshape(1, _C), nrm)
    return out[:, 0]


@functools.partial(jax.jit, static_argnames=("n_out16",))
def _segsum_sc(table, src2, dst2, ew2, cnt, n_out16):
    rpt = n_out16 // _NS
    mesh = plsc.VectorSubcoreMesh(core_axis_name="c", subcore_axis_name="s")

    @functools.partial(
        pl.kernel,
        mesh=mesh,
        out_type=jax.ShapeDtypeStruct((_NC, n_out16, _C), jnp.float32),
        scratch_types=[
            pltpu.VMEM((_CHUNK,), jnp.int32),
            pltpu.VMEM((_CHUNK,), jnp.int32),
            pltpu.VMEM((_CHUNK,), jnp.float32),
            pltpu.VMEM((_CHUNK, _C), jnp.float32),
            pltpu.VMEM((16,), jnp.int32),
            pltpu.VMEM_SHARED((n_out16, _C), jnp.float32),
            pltpu.SemaphoreType.DMA,
        ],
    )
    def k(table_h, src_h, dst_h, ew_h, cnt_h, out_h,
          src_v, dst_v, ew_v, rows_v, cnt_v, acc_s, sem):
        cid = lax.axis_index("c")
        sid = lax.axis_index("s")
        wid = sid * _NC + cid

        def zrow(r, c):
            for j in range(_C // 16):
                rows_v[r, pl.ds(j * 16, 16)] = jnp.zeros((16,), jnp.float32)
            return c
        lax.fori_loop(0, _CHUNK, zrow, 0)

        base = sid * rpt
        off = 0
        while off < rpt:
            sz = min(_CHUNK, rpt - off)
            pltpu.sync_copy(rows_v.at[pl.ds(0, sz)],
                            acc_s.at[pl.ds(base + off, sz)])
            off += sz
        plsc.subcore_barrier()

        pltpu.sync_copy(cnt_h, cnt_v)
        n_edges = cnt_v[pl.ds(0, 16)][0]
        n_grp = (n_edges + _EPG - 1) // _EPG

        def body(g, c):
            row = g * _NW + wid
            pltpu.sync_copy(src_h.at[row], src_v)
            pltpu.sync_copy(dst_h.at[row], dst_v)
            pltpu.sync_copy(ew_h.at[row], ew_v)
            pltpu.async_copy(table_h.at[src_v], rows_v, sem).wait()

            def scale(rb, cc):
                wv = ew_v[pl.ds(rb * 16, 16)]
                for j in range(16):
                    w = wv[j]
                    r = rb * 16 + j
                    for cb in range(_C // 16):
                        sl = pl.ds(cb * 16, 16)
                        rows_v[r, sl] = rows_v[r, sl] * w
                return cc
            lax.fori_loop(0, _CHUNK // 16, scale, 0)
            pltpu.sync_copy(rows_v, acc_s.at[dst_v], add=True)
            return c
        lax.fori_loop(0, n_grp, body, 0)
        plsc.subcore_barrier()

        off = 0
        while off < rpt:
            sz = min(_CHUNK, rpt - off)
            pltpu.sync_copy(acc_s.at[pl.ds(base + off, sz)],
                            out_h.at[cid, pl.ds(base + off, sz)])
            off += sz

    return k(table, src2, dst2, ew2, cnt)


def _segsum(table, src2, dst2, ew2, cnt, n_out):
    n_out16 = -(-n_out // (_NS * 8)) * (_NS * 8)
    parts = _segsum_sc(table, src2, dst2, ew2, cnt, n_out16)
    return (parts[0] + parts[1])[:n_out]


def _pad_rows(a, rows):
    pad = rows * _CHUNK - a.shape[0]
    if pad:
        a = jnp.concatenate([a, jnp.zeros((pad,), a.dtype)])
    return a.reshape(rows, _CHUNK)


def _compact_pad(mask, arrs, rows):
    pos = jnp.cumsum(mask.astype(jnp.int32)) - 1
    tgt = jnp.where(mask, pos, rows * _CHUNK)
    outs = []
    for a in arrs:
        z = jnp.zeros((rows * _CHUNK,), a.dtype)
        outs.append(z.at[tgt].set(a, mode="drop").reshape(rows, _CHUNK))
    cnt = jnp.full((16,), jnp.sum(mask, dtype=jnp.int32), jnp.int32)
    return outs, cnt


def _rows_for(n):
    return -(-n // _EPG) * _NW


def kernel(x, edge_index, edge_weight, node_pos, mlp_W, mlp_b,
           mp_Wself, mp_Wmu, pool_p):
    del node_pos
    N0 = x.shape[0]
    E = edge_weight.shape[0]
    k1 = math.ceil(0.5 * N0)
    k2 = math.ceil(0.5 * k1)

    def mus(i, xx, src2, dst2, ew2, cnt):
        n = xx.shape[0]
        xx = _mlp(xx, mlp_W[i], mlp_b[i])
        for l in range(mp_Wself.shape[1]):
            a1 = _segsum(xx, src2, dst2, ew2, cnt, n)
            a2 = _segsum(a1, src2, dst2, ew2, cnt, n)
            xx = _combine(xx, a1, a2, mp_Wself[i, l],
                          mp_Wmu[i, l, 0], mp_Wmu[i, l, 1])
        return xx

    src0, dst0 = edge_index[0], edge_index[1]
    rows_e = _rows_for(E)
    src0_2 = _pad_rows(src0, rows_e)
    dst0_2 = _pad_rows(dst0, rows_e)
    ew0_2 = _pad_rows(edge_weight, rows_e)
    cnt0 = jnp.full((16,), E, jnp.int32)

    x0 = mus(0, x, src0_2, dst0_2, ew0_2, cnt0)

    score1 = _score(x0, pool_p[0])
    _, perm1 = lax.top_k(score1, k1)
    nmap1 = jnp.full((N0,), -1, jnp.int32).at[perm1].set(
        jnp.arange(k1, dtype=jnp.int32))
    s1f = nmap1[src0]
    d1f = nmap1[dst0]
    mask1 = (s1f >= 0) & (d1f >= 0)
    s1f = jnp.where(mask1, s1f, 0)
    d1f = jnp.where(mask1, d1f, 0)

    rows_p1 = _rows_for(k1)
    g_src = _pad_rows(perm1, rows_p1)
    g_dst = _pad_rows(jnp.arange(k1, dtype=jnp.int32), rows_p1)
    g_ew = _pad_rows(score1[perm1], rows_p1)
    cnt_p1 = jnp.full((16,), k1, jnp.int32)
    x1_in = _segsum(x0, g_src, g_dst, g_ew, cnt_p1, k1)

    (e1,), cnt1 = _compact_pad(mask1, [s1f], rows_e)
    (e1d, ew1), _ = _compact_pad(mask1, [d1f, edge_weight], rows_e)

    x1 = mus(1, x1_in, e1, e1d, ew1, cnt1)

    score2 = _score(x1, pool_p[1])
    _, perm2 = lax.top_k(score2, k2)
    nmap2 = jnp.full((k1,), -1, jnp.int32).at[perm2].set(
        jnp.arange(k2, dtype=jnp.int32))
    s2f = nmap2[s1f]
    d2f = nmap2[d1f]
    mask2 = (s2f >= 0) & (d2f >= 0)
    m12 = mask1 & mask2
    s2f = jnp.where(m12, s2f, 0)
    d2f = jnp.where(m12, d2f, 0)

    rows_p2 = _rows_for(k2)
    g2_src = _pad_rows(perm2, rows_p2)
    g2_dst = _pad_rows(jnp.arange(k2, dtype=jnp.int32), rows_p2)
    g2_ew = _pad_rows(score2[perm2], rows_p2)
    cnt_p2 = jnp.full((16,), k2, jnp.int32)
    x2_in = _segsum(x1, g2_src, g2_dst, g2_ew, cnt_p2, k2)

    (c2s, c2d, c1s, c1d, c0s, c0d, cew), cnt12 = _compact_pad(
        m12, [s2f, d2f, s1f, d1f, src0, dst0, edge_weight], rows_e)

    x2 = mus(2, x2_in, c2s, c2d, cew, cnt12)
    x3 = mus(3, x2, c2s, c2d, cew, cnt12)

    rows_u1 = _rows_for(k2)
    u1_src = _pad_rows(jnp.arange(k2, dtype=jnp.int32), rows_u1)
    u1_dst = _pad_rows(perm2, rows_u1)
    u1_ew = _pad_rows(jnp.ones((k2,), jnp.float32), rows_u1)
    x4_in = _segsum(x3, u1_src, u1_dst, u1_ew, cnt_p2, k1)
    x4 = mus(4, x4_in, c1s, c1d, cew, cnt12)

    rows_u0 = _rows_for(k1)
    u0_src = _pad_rows(jnp.arange(k1, dtype=jnp.int32), rows_u0)
    u0_dst = _pad_rows(perm1, rows_u0)
    u0_ew = _pad_rows(jnp.ones((k1,), jnp.float32), rows_u0)
    x5_in = _segsum(x4, u0_src, u0_dst, u0_ew, cnt_p1, N0)
    x5 = mus(5, x5_in, c0s, c0d, cew, cnt12)
    return x5
